# baseline (speedup 1.0000x reference)
import functools

import jax
import jax.numpy as jnp
from jax import lax
from jax.experimental import pallas as pl
from jax.experimental.pallas import tpu as pltpu
from jax.experimental.pallas import tpu_sc as plsc

_NE = 8
_K = 2
_D = 1024
_I = 2816
_T = 2048
_TM = 256
_P = 6144
_NT = _P // _TM
_NW = 32


def _router_body(x_ref, r_ref, rb_ref, sma_ref, ppw_ref, wsel_ref, te_ref,
                 msc_ref):
    f32 = jnp.float32
    lane = jax.lax.broadcasted_iota(jnp.int32, (1, 128), 1)
    valid = lane < _NE
    neg = jnp.float32(-1e30)
    logits = jnp.dot(x_ref[...], r_ref[...], preferred_element_type=f32)
    t = logits.shape[0]
    lg = jnp.where(valid, logits, neg)
    m = jnp.max(lg, axis=1, keepdims=True)
    ex = jnp.where(valid, jnp.exp(lg - m), 0.0)
    se = jnp.sum(ex, axis=1, keepdims=True)
    probs = ex / se
    z = jnp.log(se) + m
    zloss = jnp.sum(z * z, axis=0, keepdims=True)[:, 0:1] / t
    bl = jnp.where(valid, logits + rb_ref[...], neg)

    def pick(cur):
        v = jnp.max(cur, axis=1, keepdims=True)
        idx = jnp.min(jnp.where(cur == v, lane, 128), axis=1, keepdims=True)
        return v, idx, jnp.where(lane == idx, neg, cur)

    _, i1, bl1 = pick(bl)
    _, i2, bl2 = pick(bl1)
    qb_alpha, _, _ = pick(bl2)
    oh1 = lane == i1
    oh2 = lane == i2
    w1 = jax.nn.sigmoid(jnp.sum(jnp.where(oh1, logits, 0.0), axis=1,
                                keepdims=True))
    w2 = jax.nn.sigmoid(jnp.sum(jnp.where(oh2, logits, 0.0), axis=1,
                                keepdims=True))
    ohb = oh1.astype(f32) + oh2.astype(f32)
    counts = jnp.sum(ohb, axis=0, keepdims=True)
    p_mean = jnp.sum(probs, axis=0, keepdims=True) / t
    tot = jnp.maximum(jnp.sum(counts, axis=1, keepdims=True), 1.0)
    frac = counts / tot
    ent = -jnp.sum(jnp.where(valid, frac * jnp.log(frac + 1e-06), 0.0),
                   axis=1, keepdims=True)
    lbl = _NE * jnp.sum(frac * _K * p_mean, axis=1, keepdims=True)
    cum = ohb
    sft = 1
    while sft < t:
        cum = cum + jnp.concatenate(
            [jnp.zeros((sft, 128), f32), cum[:-sft]], axis=0)
        sft *= 2
    excl = cum - ohb
    pad_c = jnp.where(valid, jnp.floor((counts + (_TM - 1.0)) / _TM) * _TM,
                      0.0)
    lcum = pad_c
    sft = 1
    while sft < 128:
        lcum = lcum + jnp.concatenate(
            [jnp.zeros((1, sft), f32), lcum[:, :-sft]], axis=1)
        sft *= 2
    ps = lcum - pad_c
    pp1 = jnp.sum(jnp.where(oh1, ps + excl, 0.0), axis=1, keepdims=True)
    pp2 = jnp.sum(jnp.where(oh2, ps + excl, 0.0), axis=1, keepdims=True)
    l0 = (lane == 0).astype(f32)
    l1 = (lane == 1).astype(f32)
    sma_ref[...] = logits - qb_alpha
    ppw_ref[...] = (pp1 * l0 + pp2 * l1).astype(jnp.int32)
    wsel_ref[...] = w1 * l0 + w2 * l1
    g256 = (lane * _TM).astype(f32)
    te_row = jnp.zeros((1, 128), jnp.int32)
    for e in range(_NE):
        ce = jax.lax.slice(lcum, (0, e), (1, e + 1))
        te_row = te_row + (g256 >= ce).astype(jnp.int32)
    total = jax.lax.slice(lcum, (0, _NE - 1), (1, _NE))
    valid_row = (g256 < total).astype(jnp.int32)
    te_ref[...] = jnp.concatenate(
        [jnp.minimum(te_row, _NE - 1), valid_row,
         jnp.zeros((6, 128), jnp.int32)], axis=0)
    row0 = lbl * l0 + zloss * l1 + ent * (lane == 2).astype(f32)
    msc_ref[...] = jnp.concatenate(
        [row0, counts, jnp.zeros((6, 128), f32)], axis=0)


_IJ = 1408


def _gmm_body(tei_ref, xs_ref, w_ref, wg_ref, wu_ref, wd_ref, o_ref,
              acc_ref, wgc_ref, wuc_ref):
    j = pl.program_id(0)
    i = pl.program_id(1)
    fresh = (i == 0) | (tei_ref[i] != tei_ref[jnp.maximum(i - 1, 0)])

    @pl.when(fresh)
    def _():
        wgc_ref[...] = wg_ref[0].astype(jnp.bfloat16)
        wuc_ref[...] = wu_ref[0].astype(jnp.bfloat16)

    @pl.when(tei_ref[_NT + i] == 1)
    def _():
        xg = xs_ref[...].astype(jnp.bfloat16)
        g = jnp.dot(xg, wgc_ref[...], preferred_element_type=jnp.float32)
        u = jnp.dot(xg, wuc_ref[...], preferred_element_type=jnp.float32)
        h = (g * jax.nn.sigmoid(g) * u).astype(jnp.bfloat16)
        o = jnp.dot(h, wd_ref[0].astype(jnp.bfloat16),
                    preferred_element_type=jnp.float32)
        contrib = o * w_ref[:, 0:1]
        rows = pl.ds(i * _TM, _TM)

        @pl.when(j == 0)
        def _():
            acc_ref[rows, :] = contrib.astype(jnp.bfloat16)

        @pl.when(j == 1)
        def _():
            o_ref[...] = acc_ref[rows, :].astype(jnp.float32) + contrib


def _add_body(a_ref, b_ref, o_ref):
    o_ref[...] = a_ref[0] + b_ref[0]


def _sc_gather(x_flat, tok_pad):
    mesh = plsc.VectorSubcoreMesh(core_axis_name="c", subcore_axis_name="s")

    @functools.partial(
        pl.kernel, mesh=mesh,
        out_type=jax.ShapeDtypeStruct((_P, _D), jnp.float32),
        scratch_types=[
            pltpu.VMEM((_P // _NW,), jnp.int32),
            pltpu.VMEM((48, _D), jnp.float32),
            pltpu.VMEM((48, _D), jnp.float32),
            pltpu.SemaphoreType.DMA,
            pltpu.SemaphoreType.DMA,
            pltpu.SemaphoreType.DMA,
            pltpu.SemaphoreType.DMA,
        ],
    )
    def body(x_hbm, idx_hbm, out_hbm, idx_v, rows_a, rows_b, ga, gb, wa, wb):
        wid = lax.axis_index("s") * 2 + lax.axis_index("c")
        n = _P // _NW
        cs = 48
        nc = n // cs
        base = wid * n
        pltpu.sync_copy(idx_hbm.at[pl.ds(base, n)], idx_v)
        bufs = (rows_a, rows_b)
        gsems = (ga, gb)
        wsems = (wa, wb)
        gathers = [None, None]
        writes = [None, None]
        gathers[0] = pltpu.async_copy(
            x_hbm.at[idx_v.at[pl.ds(0, cs)]], bufs[0], gsems[0])
        for c in range(nc):
            cur = c % 2
            nxt = (c + 1) % 2
            if c + 1 < nc:
                if writes[nxt] is not None:
                    writes[nxt].wait()
                gathers[nxt] = pltpu.async_copy(
                    x_hbm.at[idx_v.at[pl.ds((c + 1) * cs, cs)]],
                    bufs[nxt], gsems[nxt])
            gathers[cur].wait()
            writes[cur] = pltpu.async_copy(
                bufs[cur], out_hbm.at[pl.ds(base + c * cs, cs)], wsems[cur])
        writes[0].wait()
        writes[1].wait()

    return body(x_flat, tok_pad)


def _sc_combine(o_sorted, p0, p1):
    mesh = plsc.VectorSubcoreMesh(core_axis_name="c", subcore_axis_name="s")

    @functools.partial(
        pl.kernel, mesh=mesh,
        out_type=jax.ShapeDtypeStruct((2, _T, _D), jnp.float32),
        scratch_types=[
            pltpu.VMEM((_T // _NW,), jnp.int32),
            pltpu.VMEM((32, _D), jnp.float32),
            pltpu.SemaphoreType.DMA,
        ],
    )
    def body(osrc_hbm, p0_hbm, p1_hbm, out_hbm, idx_v, rows_v, sem):
        wid = lax.axis_index("s") * 2 + lax.axis_index("c")
        n = _T // _NW
        base = wid * n
        for half in range(2):
            p_hbm = p0_hbm if half == 0 else p1_hbm
            pltpu.sync_copy(p_hbm.at[pl.ds(base, n)], idx_v)
            for c in range(n // 32):
                pltpu.async_copy(
                    osrc_hbm.at[idx_v.at[pl.ds(c * 32, 32)]], rows_v,
                    sem).wait()
                pltpu.sync_copy(
                    rows_v, out_hbm.at[half, pl.ds(base + c * 32, 32)])

    return body(o_sorted, p0, p1)


def kernel(x, router, router_bias, w_gate_up, w_down):
    b, s, d = x.shape
    t = b * s
    x_flat = x.reshape(t, d)

    router_p = jnp.zeros((d, 128), jnp.float32).at[:, :_NE].set(router)
    rb_p = jnp.zeros((1, 128), jnp.float32).at[0, :_NE].set(
        jax.lax.stop_gradient(router_bias))
    sma, ppw, wsel, te_o, msc = pl.pallas_call(
        _router_body,
        out_shape=[
            jax.ShapeDtypeStruct((t, 128), jnp.float32),
            jax.ShapeDtypeStruct((t, 128), jnp.int32),
            jax.ShapeDtypeStruct((t, 128), jnp.float32),
            jax.ShapeDtypeStruct((8, 128), jnp.int32),
            jax.ShapeDtypeStruct((8, 128), jnp.float32),
        ],
    )(x_flat, router_p, rb_p)
    load_balancing_loss = msc[0, 0]
    router_z_loss = msc[0, 1]
    routing_entropy = msc[0, 2]
    expert_counts = msc[1, :_NE]
    tile_expert = jnp.concatenate([te_o[0, :_NT], te_o[1, :_NT]])
    qb_count = max(1, t * _K // _NE)
    topv, _ = jax.lax.top_k(sma[:, :_NE].T, qb_count)
    qb_beta = topv[:, -1]

    pos = ppw[:, :_K]
    pp = pos.reshape(t * _K)
    wf = wsel[:, :_K].reshape(t * _K)
    tok_pad = (jnp.arange(_P, dtype=jnp.int32) % t).at[pp].set(
        (jnp.arange(t * _K, dtype=jnp.int32) // _K))
    w_pad = jnp.zeros((_P, 128), jnp.float32).at[pp, 0].set(wf)

    x_sorted = _sc_gather(x_flat, tok_pad)

    o_sorted = pl.pallas_call(
        _gmm_body,
        grid_spec=pltpu.PrefetchScalarGridSpec(
            num_scalar_prefetch=1,
            grid=(2, _NT),
            in_specs=[
                pl.BlockSpec((_TM, _D), lambda j, i, te: (i, 0)),
                pl.BlockSpec((_TM, 128), lambda j, i, te: (i, 0)),
                pl.BlockSpec((1, _D, _IJ), lambda j, i, te: (te[i], 0, j)),
                pl.BlockSpec((1, _D, _IJ), lambda j, i, te: (te[i], 0, j + 2)),
                pl.BlockSpec((1, _IJ, _D), lambda j, i, te: (te[i], j, 0)),
            ],
            out_specs=pl.BlockSpec((_TM, _D), lambda j, i, te: (i, 0)),
            scratch_shapes=[
                pltpu.VMEM((_P, _D), jnp.bfloat16),
                pltpu.VMEM((_D, _IJ), jnp.bfloat16),
                pltpu.VMEM((_D, _IJ), jnp.bfloat16),
            ],
        ),
        out_shape=jax.ShapeDtypeStruct((_P, _D), jnp.float32),
    )(tile_expert, x_sorted, w_pad, w_gate_up, w_gate_up, w_down)

    planes = _sc_combine(o_sorted, pos[:, 0], pos[:, 1])

    routed = pl.pallas_call(
        _add_body,
        grid=(4,),
        in_specs=[
            pl.BlockSpec((1, t // 4, d), lambda i: (0, i, 0)),
            pl.BlockSpec((1, t // 4, d), lambda i: (1, i, 0)),
        ],
        out_specs=pl.BlockSpec((t // 4, d), lambda i: (i, 0)),
        out_shape=jax.ShapeDtypeStruct((t, d), jnp.float32),
    )(planes, planes)

    return (routed.reshape(b, s, d), load_balancing_loss, router_z_loss,
            routing_entropy, expert_counts, qb_beta)


# speedup vs baseline: 1.0250x; 1.0250x over previous
import functools

import jax
import jax.numpy as jnp
from jax import lax
from jax.experimental import pallas as pl
from jax.experimental.pallas import tpu as pltpu
from jax.experimental.pallas import tpu_sc as plsc

_NE = 8
_K = 2
_D = 1024
_I = 2816
_T = 2048
_TM = 256
_P = 6144
_NT = _P // _TM
_NW = 32


def _router_body(x_ref, r_ref, rb_ref, sma_ref, ppw_ref, wsel_ref, te_ref,
                 msc_ref):
    f32 = jnp.float32
    lane = jax.lax.broadcasted_iota(jnp.int32, (1, 128), 1)
    valid = lane < _NE
    neg = jnp.float32(-1e30)
    logits = jnp.dot(x_ref[...], r_ref[...], preferred_element_type=f32)
    t = logits.shape[0]
    lg = jnp.where(valid, logits, neg)
    m = jnp.max(lg, axis=1, keepdims=True)
    ex = jnp.where(valid, jnp.exp(lg - m), 0.0)
    se = jnp.sum(ex, axis=1, keepdims=True)
    probs = ex / se
    z = jnp.log(se) + m
    zloss = jnp.sum(z * z, axis=0, keepdims=True)[:, 0:1] / t
    bl = jnp.where(valid, logits + rb_ref[...], neg)

    def pick(cur):
        v = jnp.max(cur, axis=1, keepdims=True)
        idx = jnp.min(jnp.where(cur == v, lane, 128), axis=1, keepdims=True)
        return v, idx, jnp.where(lane == idx, neg, cur)

    _, i1, bl1 = pick(bl)
    _, i2, bl2 = pick(bl1)
    qb_alpha, _, _ = pick(bl2)
    oh1 = lane == i1
    oh2 = lane == i2
    w1 = jax.nn.sigmoid(jnp.sum(jnp.where(oh1, logits, 0.0), axis=1,
                                keepdims=True))
    w2 = jax.nn.sigmoid(jnp.sum(jnp.where(oh2, logits, 0.0), axis=1,
                                keepdims=True))
    ohb = oh1.astype(f32) + oh2.astype(f32)
    counts = jnp.sum(ohb, axis=0, keepdims=True)
    p_mean = jnp.sum(probs, axis=0, keepdims=True) / t
    tot = jnp.maximum(jnp.sum(counts, axis=1, keepdims=True), 1.0)
    frac = counts / tot
    ent = -jnp.sum(jnp.where(valid, frac * jnp.log(frac + 1e-06), 0.0),
                   axis=1, keepdims=True)
    lbl = _NE * jnp.sum(frac * _K * p_mean, axis=1, keepdims=True)
    cum = ohb
    sft = 1
    while sft < t:
        cum = cum + jnp.concatenate(
            [jnp.zeros((sft, 128), f32), cum[:-sft]], axis=0)
        sft *= 2
    excl = cum - ohb
    pad_c = jnp.where(valid, jnp.floor((counts + (_TM - 1.0)) / _TM) * _TM,
                      0.0)
    lcum = pad_c
    sft = 1
    while sft < 128:
        lcum = lcum + jnp.concatenate(
            [jnp.zeros((1, sft), f32), lcum[:, :-sft]], axis=1)
        sft *= 2
    ps = lcum - pad_c
    pp1 = jnp.sum(jnp.where(oh1, ps + excl, 0.0), axis=1, keepdims=True)
    pp2 = jnp.sum(jnp.where(oh2, ps + excl, 0.0), axis=1, keepdims=True)
    l0 = (lane == 0).astype(f32)
    l1 = (lane == 1).astype(f32)
    sma_ref[...] = logits - qb_alpha
    ppw_ref[...] = (pp1 * l0 + pp2 * l1).astype(jnp.int32)
    wsel_ref[...] = w1 * l0 + w2 * l1
    g256 = (lane * _TM).astype(f32)
    te_row = jnp.zeros((1, 128), jnp.int32)
    for e in range(_NE):
        ce = jax.lax.slice(lcum, (0, e), (1, e + 1))
        te_row = te_row + (g256 >= ce).astype(jnp.int32)
    total = jax.lax.slice(lcum, (0, _NE - 1), (1, _NE))
    valid_row = (g256 < total).astype(jnp.int32)
    te_ref[...] = jnp.concatenate(
        [jnp.minimum(te_row, _NE - 1), valid_row,
         jnp.zeros((6, 128), jnp.int32)], axis=0)
    row0 = lbl * l0 + zloss * l1 + ent * (lane == 2).astype(f32)
    msc_ref[...] = jnp.concatenate(
        [row0, counts, jnp.zeros((6, 128), f32)], axis=0)


_IJ = 1408


def _gmm_body(tei_ref, xs_ref, w_ref, wg_ref, wu_ref, wd_ref, o_ref,
              acc_ref):
    j = pl.program_id(0)
    i = pl.program_id(1)

    @pl.when(tei_ref[_NT + i] == 1)
    def _():
        xg = xs_ref[...]
        g = jnp.dot(xg, wg_ref[0], preferred_element_type=jnp.float32)
        u = jnp.dot(xg, wu_ref[0], preferred_element_type=jnp.float32)
        h = g * jax.nn.sigmoid(g) * u
        o = jnp.dot(h, wd_ref[0], preferred_element_type=jnp.float32)
        contrib = o * w_ref[:, 0:1]
        rows = pl.ds(i * _TM, _TM)

        @pl.when(j == 0)
        def _():
            acc_ref[rows, :] = contrib.astype(jnp.bfloat16)

        @pl.when(j == 1)
        def _():
            o_ref[...] = acc_ref[rows, :].astype(jnp.float32) + contrib


def _add_body(a_ref, b_ref, o_ref):
    o_ref[...] = a_ref[0] + b_ref[0]


def _sc_gather(x_flat, tok_pad):
    mesh = plsc.VectorSubcoreMesh(core_axis_name="c", subcore_axis_name="s")

    @functools.partial(
        pl.kernel, mesh=mesh,
        out_type=jax.ShapeDtypeStruct((_P, _D), jnp.float32),
        scratch_types=[
            pltpu.VMEM((_P // _NW,), jnp.int32),
            pltpu.VMEM((48, _D), jnp.float32),
            pltpu.VMEM((48, _D), jnp.float32),
            pltpu.SemaphoreType.DMA,
            pltpu.SemaphoreType.DMA,
            pltpu.SemaphoreType.DMA,
            pltpu.SemaphoreType.DMA,
        ],
    )
    def body(x_hbm, idx_hbm, out_hbm, idx_v, rows_a, rows_b, ga, gb, wa, wb):
        wid = lax.axis_index("s") * 2 + lax.axis_index("c")
        n = _P // _NW
        cs = 48
        nc = n // cs
        base = wid * n
        pltpu.sync_copy(idx_hbm.at[pl.ds(base, n)], idx_v)
        bufs = (rows_a, rows_b)
        gsems = (ga, gb)
        wsems = (wa, wb)
        gathers = [None, None]
        writes = [None, None]
        gathers[0] = pltpu.async_copy(
            x_hbm.at[idx_v.at[pl.ds(0, cs)]], bufs[0], gsems[0])
        for c in range(nc):
            cur = c % 2
            nxt = (c + 1) % 2
            if c + 1 < nc:
                if writes[nxt] is not None:
                    writes[nxt].wait()
                gathers[nxt] = pltpu.async_copy(
                    x_hbm.at[idx_v.at[pl.ds((c + 1) * cs, cs)]],
                    bufs[nxt], gsems[nxt])
            gathers[cur].wait()
            writes[cur] = pltpu.async_copy(
                bufs[cur], out_hbm.at[pl.ds(base + c * cs, cs)], wsems[cur])
        writes[0].wait()
        writes[1].wait()

    return body(x_flat, tok_pad)


def _sc_combine(o_sorted, p0, p1):
    mesh = plsc.VectorSubcoreMesh(core_axis_name="c", subcore_axis_name="s")

    @functools.partial(
        pl.kernel, mesh=mesh,
        out_type=jax.ShapeDtypeStruct((2, _T, _D), jnp.float32),
        scratch_types=[
            pltpu.VMEM((_T // _NW,), jnp.int32),
            pltpu.VMEM((32, _D), jnp.float32),
            pltpu.SemaphoreType.DMA,
        ],
    )
    def body(osrc_hbm, p0_hbm, p1_hbm, out_hbm, idx_v, rows_v, sem):
        wid = lax.axis_index("s") * 2 + lax.axis_index("c")
        n = _T // _NW
        base = wid * n
        for half in range(2):
            p_hbm = p0_hbm if half == 0 else p1_hbm
            pltpu.sync_copy(p_hbm.at[pl.ds(base, n)], idx_v)
            for c in range(n // 32):
                pltpu.async_copy(
                    osrc_hbm.at[idx_v.at[pl.ds(c * 32, 32)]], rows_v,
                    sem).wait()
                pltpu.sync_copy(
                    rows_v, out_hbm.at[half, pl.ds(base + c * 32, 32)])

    return body(o_sorted, p0, p1)


def kernel(x, router, router_bias, w_gate_up, w_down):
    b, s, d = x.shape
    t = b * s
    x_flat = x.reshape(t, d)

    router_p = jnp.zeros((d, 128), jnp.float32).at[:, :_NE].set(router)
    rb_p = jnp.zeros((1, 128), jnp.float32).at[0, :_NE].set(
        jax.lax.stop_gradient(router_bias))
    sma, ppw, wsel, te_o, msc = pl.pallas_call(
        _router_body,
        out_shape=[
            jax.ShapeDtypeStruct((t, 128), jnp.float32),
            jax.ShapeDtypeStruct((t, 128), jnp.int32),
            jax.ShapeDtypeStruct((t, 128), jnp.float32),
            jax.ShapeDtypeStruct((8, 128), jnp.int32),
            jax.ShapeDtypeStruct((8, 128), jnp.float32),
        ],
    )(x_flat, router_p, rb_p)
    load_balancing_loss = msc[0, 0]
    router_z_loss = msc[0, 1]
    routing_entropy = msc[0, 2]
    expert_counts = msc[1, :_NE]
    tile_expert = jnp.concatenate([te_o[0, :_NT], te_o[1, :_NT]])
    qb_count = max(1, t * _K // _NE)
    topv, _ = jax.lax.top_k(sma[:, :_NE].T, qb_count)
    qb_beta = topv[:, -1]

    pos = ppw[:, :_K]
    pp = pos.reshape(t * _K)
    wf = wsel[:, :_K].reshape(t * _K)
    tok_pad = (jnp.arange(_P, dtype=jnp.int32) % t).at[pp].set(
        (jnp.arange(t * _K, dtype=jnp.int32) // _K))
    w_pad = jnp.zeros((_P, 128), jnp.float32).at[pp, 0].set(wf)

    x_sorted = _sc_gather(x_flat, tok_pad)

    o_sorted = pl.pallas_call(
        _gmm_body,
        grid_spec=pltpu.PrefetchScalarGridSpec(
            num_scalar_prefetch=1,
            grid=(2, _NT),
            in_specs=[
                pl.BlockSpec((_TM, _D), lambda j, i, te: (i, 0)),
                pl.BlockSpec((_TM, 128), lambda j, i, te: (i, 0)),
                pl.BlockSpec((1, _D, _IJ), lambda j, i, te: (te[i], 0, j)),
                pl.BlockSpec((1, _D, _IJ), lambda j, i, te: (te[i], 0, j + 2)),
                pl.BlockSpec((1, _IJ, _D), lambda j, i, te: (te[i], j, 0)),
            ],
            out_specs=pl.BlockSpec((_TM, _D), lambda j, i, te: (i, 0)),
            scratch_shapes=[pltpu.VMEM((_P, _D), jnp.bfloat16)],
        ),
        out_shape=jax.ShapeDtypeStruct((_P, _D), jnp.float32),
    )(tile_expert, x_sorted, w_pad, w_gate_up, w_gate_up, w_down)

    planes = _sc_combine(o_sorted, pos[:, 0], pos[:, 1])

    routed = pl.pallas_call(
        _add_body,
        grid=(4,),
        in_specs=[
            pl.BlockSpec((1, t // 4, d), lambda i: (0, i, 0)),
            pl.BlockSpec((1, t // 4, d), lambda i: (1, i, 0)),
        ],
        out_specs=pl.BlockSpec((t // 4, d), lambda i: (i, 0)),
        out_shape=jax.ShapeDtypeStruct((t, d), jnp.float32),
    )(planes, planes)

    return (routed.reshape(b, s, d), load_balancing_loss, router_z_loss,
            routing_entropy, expert_counts, qb_beta)
